# baseline (speedup 1.0000x reference)
import jax
import jax.numpy as jnp
from jax import lax
from jax.experimental import pallas as pl
from jax.experimental.pallas import tpu as pltpu

B, N, D_FEAT, D_COORD, D_MODEL, K = 4, 2048, 128, 3, 256, 16
TN = 128
NT = N // TN
SCALE = float(D_MODEL) ** (-0.5)
ROWS = K * TN


def _fold_kernel(w1_ref, b1_ref, w3_ref, p_ref, pb_ref):
    dn = (((1,), (0,)), ((), ()))
    p_ref[...] = lax.dot_general(w1_ref[...], w3_ref[...], dn,
                                 precision=lax.Precision.HIGHEST)
    pb_ref[...] = lax.dot_general(b1_ref[...], w3_ref[...], dn,
                                  precision=lax.Precision.HIGHEST)


def _bf(x):
    return x.astype(jnp.bfloat16)


def _mm(a, b):
    return lax.dot_general(a, b, (((1,), (0,)), ((), ())),
                           preferred_element_type=jnp.float32)


def _main_kernel(ftab_ref, coordT_ref, ctk_ref, ctm_ref, ftile_ref,
                 wq_ref, bq_ref, wkv_ref, bkv_ref,
                 tw1_ref, tb1_ref, tw2_ref, tb2_ref,
                 gw1_ref, gb1_ref, gw2_ref, gb2_ref,
                 w2_ref, b2_ref, out_ref, oh_ref):
    g = _mm(oh_ref[...], ftab_ref[0])

    ct = ctk_ref[0]
    cfT = coordT_ref[0]

    sq_t = jnp.sum(ct * ct, axis=1, keepdims=True)
    sq_f = jnp.sum(cfT * cfT, axis=0, keepdims=True)
    ctr = ct.astype(jnp.bfloat16).astype(jnp.float32)
    cfr = cfT.astype(jnp.bfloat16).astype(jnp.float32)
    cross = (ctr[:, 0:1] * cfr[0:1, :]
             + ctr[:, 1:2] * cfr[1:2, :]
             + ctr[:, 2:3] * cfr[2:3, :])
    d2 = jnp.maximum(sq_t + sq_f - 2.0 * cross, 0.0)

    lane = lax.broadcasted_iota(jnp.int32, (TN, N), 1)
    key = lax.bitcast_convert_type(
        ((lax.bitcast_convert_type(d2, jnp.int32) & jnp.int32(~0x7FF))
         | lane) + jnp.int32(0x08000000), jnp.float32)
    m = jnp.min(key, axis=1, keepdims=True)
    oh_ref[0:TN, :] = (key == m).astype(jnp.bfloat16)
    for r in range(1, K):
        m = jnp.min(jnp.where(key <= m, jnp.float32(jnp.inf), key),
                    axis=1, keepdims=True)
        oh_ref[r * TN:(r + 1) * TN, :] = (key == m).astype(jnp.bfloat16)

    gfeat = g[:, 0:D_FEAT]
    gcoord = g[:, D_FEAT:D_FEAT + 3] + g[:, D_FEAT + 3:D_FEAT + 6]

    kv = _mm(_bf(gfeat), wkv_ref[...]) + bkv_ref[...]
    gk = kv[:, 0:D_MODEL]
    gv = kv[:, D_MODEL:2 * D_MODEL]

    ctm = ctm_ref[0]
    ctrep = jnp.broadcast_to(ctm[None], (K, TN, 3)).reshape(ROWS, 3)
    rel = ctrep - gcoord
    pe = jnp.maximum(_mm(_bf(rel), tw1_ref[...]) + tb1_ref[...], 0.0)
    pe = _mm(_bf(pe), tw2_ref[...]) + tb2_ref[...]

    q = _mm(_bf(ftile_ref[0]), wq_ref[...]) + bq_ref[...]
    qrep = jnp.broadcast_to(q[None], (K, TN, D_MODEL)).reshape(ROWS, D_MODEL)
    a = qrep - gk + pe
    a = jnp.maximum(_mm(_bf(a), gw1_ref[...]) + gb1_ref[...], 0.0)
    a = _mm(_bf(a), gw2_ref[...]) + gb2_ref[...]
    z = (a * SCALE).reshape(K, TN, D_MODEL)
    z = z - jnp.max(z, axis=0, keepdims=True)
    e = jnp.exp(z)
    attn = e / jnp.sum(e, axis=0, keepdims=True)

    vpe = (gv + pe).reshape(K, TN, D_MODEL)
    feat = jnp.sum(attn * vpe, axis=0)
    out_ref[0] = _mm(_bf(feat), w2_ref[...]) + b2_ref[...] + ftile_ref[0]


def kernel(feature, coord, w1, b1, w2, b2, gw1, gb1, gw2, gb2,
           tw1, tb1, tw2, tb2, wq, wk, wv):
    f32 = jnp.float32
    w3 = jnp.concatenate([wq, wk, wv], axis=1)
    p, pb = pl.pallas_call(
        _fold_kernel,
        out_shape=[jax.ShapeDtypeStruct((D_FEAT, 3 * D_MODEL), f32),
                   jax.ShapeDtypeStruct((1, 3 * D_MODEL), f32)],
    )(w1, b1.reshape(1, D_MODEL), w3)
    wq_f = _bf(p[:, 0:D_MODEL])
    wkv_f = _bf(p[:, D_MODEL:])
    bq_f = pb[:, 0:D_MODEL]
    bkv_f = pb[:, D_MODEL:]

    chi = _bf(coord)
    clo = _bf(coord - chi.astype(f32))
    ftab = jnp.concatenate([_bf(feature), chi, clo], axis=-1)
    coordT = coord.transpose(0, 2, 1)

    full = lambda shape: pl.BlockSpec(shape, lambda b, s: (0,) * len(shape))
    b_only = lambda shape: pl.BlockSpec(shape, lambda b, s: (b,) + (0,) * (len(shape) - 1))
    tile_k = lambda c: pl.BlockSpec((1, TN, c),
                                    lambda b, s: (b, jnp.minimum(s, NT - 1), 0))
    tile_m = lambda c: pl.BlockSpec((1, TN, c),
                                    lambda b, s: (b, jnp.maximum(s - 1, 0), 0))

    out = pl.pallas_call(
        _main_kernel,
        grid=(B, NT + 1),
        in_specs=[
            b_only((1, N, D_FEAT + 6)),
            b_only((1, 3, N)),
            tile_k(3),
            tile_m(3),
            tile_m(D_FEAT),
            full((D_FEAT, D_MODEL)),
            full((1, D_MODEL)),
            full((D_FEAT, 2 * D_MODEL)),
            full((1, 2 * D_MODEL)),
            full((3, D_MODEL)), full((1, D_MODEL)),
            full((D_MODEL, D_MODEL)), full((1, D_MODEL)),
            full((D_MODEL, D_MODEL)), full((1, D_MODEL)),
            full((D_MODEL, D_MODEL)), full((1, D_MODEL)),
            full((D_MODEL, D_FEAT)), full((1, D_FEAT)),
        ],
        out_specs=tile_m(D_FEAT),
        out_shape=jax.ShapeDtypeStruct((B, N, D_FEAT), f32),
        scratch_shapes=[pltpu.VMEM((ROWS, N), jnp.bfloat16)],
        compiler_params=pltpu.CompilerParams(
            dimension_semantics=("parallel", "arbitrary")),
    )(ftab, coordT, coord, coord, feature,
      wq_f, bq_f, wkv_f, bkv_f,
      _bf(tw1), tb1.reshape(1, D_MODEL), _bf(tw2), tb2.reshape(1, D_MODEL),
      _bf(gw1), gb1.reshape(1, D_MODEL), _bf(gw2), gb2.reshape(1, D_MODEL),
      _bf(w2), b2.reshape(1, D_FEAT))
    return (out, coord)


# speedup vs baseline: 1.0371x; 1.0371x over previous
import jax
import jax.numpy as jnp
from jax import lax
from jax.experimental import pallas as pl
from jax.experimental.pallas import tpu as pltpu

B, N, D_FEAT, D_COORD, D_MODEL, K = 4, 2048, 128, 3, 256, 16
TN = 128
NT = N // TN
SCALE = float(D_MODEL) ** (-0.5)
ROWS = K * TN


def _fold_kernel(w1_ref, b1_ref, w3_ref, p_ref, pb_ref):
    dn = (((1,), (0,)), ((), ()))
    p_ref[...] = lax.dot_general(w1_ref[...], w3_ref[...], dn,
                                 precision=lax.Precision.HIGHEST)
    pb_ref[...] = lax.dot_general(b1_ref[...], w3_ref[...], dn,
                                  precision=lax.Precision.HIGHEST)


def _bf(x):
    return x.astype(jnp.bfloat16)


def _mm(a, b):
    return lax.dot_general(a, b, (((1,), (0,)), ((), ())),
                           preferred_element_type=jnp.float32)


def _main_kernel(ftab_ref, coordT_ref, ctk_ref, ctm_ref, ftile_ref,
                 wq_ref, bq_ref, wkv_ref, bkv_ref,
                 tw1_ref, tb1_ref, tw2_ref, tb2_ref,
                 gw1_ref, gb1_ref, gw2_ref, gb2_ref,
                 w2_ref, b2_ref, out_ref, oh_ref):
    g = _mm(oh_ref[...], ftab_ref[0])

    ct = ctk_ref[0]
    cfT = coordT_ref[0]

    sq_t = jnp.sum(ct * ct, axis=1, keepdims=True)
    sq_f = jnp.sum(cfT * cfT, axis=0, keepdims=True)
    cross = _mm(_bf(ct), _bf(cfT))
    d2 = jnp.maximum(sq_t + sq_f - 2.0 * cross, 0.0)

    lane = lax.broadcasted_iota(jnp.int32, (TN, N), 1)
    key = lax.bitcast_convert_type(
        ((lax.bitcast_convert_type(d2, jnp.int32) & jnp.int32(~0x7FF))
         | lane) + jnp.int32(0x08000000), jnp.float32)
    for r in range(K):
        m = jnp.min(key, axis=1, keepdims=True)
        sel = key == m
        oh_ref[r * TN:(r + 1) * TN, :] = sel.astype(jnp.bfloat16)
        key = jnp.where(sel, jnp.float32(jnp.inf), key)

    gfeat = g[:, 0:D_FEAT]
    gcoord = g[:, D_FEAT:D_FEAT + 3] + g[:, D_FEAT + 3:D_FEAT + 6]

    kv = _mm(_bf(gfeat), wkv_ref[...]) + bkv_ref[...]
    gk = kv[:, 0:D_MODEL]
    gv = kv[:, D_MODEL:2 * D_MODEL]

    ctm = ctm_ref[0]
    ctrep = jnp.broadcast_to(ctm[None], (K, TN, 3)).reshape(ROWS, 3)
    rel = ctrep - gcoord
    pe = jnp.maximum(_mm(_bf(rel), tw1_ref[...]) + tb1_ref[...], 0.0)
    pe = _mm(_bf(pe), tw2_ref[...]) + tb2_ref[...]

    q = _mm(_bf(ftile_ref[0]), wq_ref[...]) + bq_ref[...]
    qrep = jnp.broadcast_to(q[None], (K, TN, D_MODEL)).reshape(ROWS, D_MODEL)
    a = qrep - gk + pe
    a = jnp.maximum(_mm(_bf(a), gw1_ref[...]) + gb1_ref[...], 0.0)
    a = _mm(_bf(a), gw2_ref[...]) + gb2_ref[...]
    z = (a * SCALE).reshape(K, TN, D_MODEL)
    z = z - jnp.max(z, axis=0, keepdims=True)
    e = jnp.exp(z)
    attn = e / jnp.sum(e, axis=0, keepdims=True)

    vpe = (gv + pe).reshape(K, TN, D_MODEL)
    feat = jnp.sum(attn * vpe, axis=0)
    out_ref[0] = _mm(_bf(feat), w2_ref[...]) + b2_ref[...] + ftile_ref[0]


def kernel(feature, coord, w1, b1, w2, b2, gw1, gb1, gw2, gb2,
           tw1, tb1, tw2, tb2, wq, wk, wv):
    f32 = jnp.float32
    w3 = jnp.concatenate([wq, wk, wv], axis=1)
    p, pb = pl.pallas_call(
        _fold_kernel,
        out_shape=[jax.ShapeDtypeStruct((D_FEAT, 3 * D_MODEL), f32),
                   jax.ShapeDtypeStruct((1, 3 * D_MODEL), f32)],
    )(w1, b1.reshape(1, D_MODEL), w3)
    wq_f = _bf(p[:, 0:D_MODEL])
    wkv_f = _bf(p[:, D_MODEL:])
    bq_f = pb[:, 0:D_MODEL]
    bkv_f = pb[:, D_MODEL:]

    chi = _bf(coord)
    clo = _bf(coord - chi.astype(f32))
    ftab = jnp.concatenate([_bf(feature), chi, clo], axis=-1)
    coordT = coord.transpose(0, 2, 1)

    full = lambda shape: pl.BlockSpec(shape, lambda b, s: (0,) * len(shape))
    b_only = lambda shape: pl.BlockSpec(shape, lambda b, s: (b,) + (0,) * (len(shape) - 1))
    tile_k = lambda c: pl.BlockSpec((1, TN, c),
                                    lambda b, s: (b, jnp.minimum(s, NT - 1), 0))
    tile_m = lambda c: pl.BlockSpec((1, TN, c),
                                    lambda b, s: (b, jnp.maximum(s - 1, 0), 0))

    out = pl.pallas_call(
        _main_kernel,
        grid=(B, NT + 1),
        in_specs=[
            b_only((1, N, D_FEAT + 6)),
            b_only((1, 3, N)),
            tile_k(3),
            tile_m(3),
            tile_m(D_FEAT),
            full((D_FEAT, D_MODEL)),
            full((1, D_MODEL)),
            full((D_FEAT, 2 * D_MODEL)),
            full((1, 2 * D_MODEL)),
            full((3, D_MODEL)), full((1, D_MODEL)),
            full((D_MODEL, D_MODEL)), full((1, D_MODEL)),
            full((D_MODEL, D_MODEL)), full((1, D_MODEL)),
            full((D_MODEL, D_MODEL)), full((1, D_MODEL)),
            full((D_MODEL, D_FEAT)), full((1, D_FEAT)),
        ],
        out_specs=tile_m(D_FEAT),
        out_shape=jax.ShapeDtypeStruct((B, N, D_FEAT), f32),
        scratch_shapes=[pltpu.VMEM((ROWS, N), jnp.bfloat16)],
        compiler_params=pltpu.CompilerParams(
            dimension_semantics=("parallel", "arbitrary")),
    )(ftab, coordT, coord, coord, feature,
      wq_f, bq_f, wkv_f, bkv_f,
      _bf(tw1), tb1.reshape(1, D_MODEL), _bf(tw2), tb2.reshape(1, D_MODEL),
      _bf(gw1), gb1.reshape(1, D_MODEL), _bf(gw2), gb2.reshape(1, D_MODEL),
      _bf(w2), b2.reshape(1, D_FEAT))
    return (out, coord)
